# baseline (speedup 1.0000x reference)
import functools

import jax
import jax.numpy as jnp
from jax import lax
from jax.experimental import pallas as pl
from jax.experimental.pallas import tpu as pltpu
from jax.experimental.pallas import tpu_sc as plsc

E = 1_600_000
K = 20
OUT_D = 42
NW = 32
EPW = E // NW
CHUNK = 400
NCHUNK = EPW // CHUNK
GROUPS = CHUNK // 16

_SMAGIC = float(1.5 * 2**23)
_INV2PI = 0.15915494309189535
_TPI_HI = 6.28125
_TPI_LO = 0.0028353071795864769
_C0 = 9.99999995e-01
_C1 = -1.66666646e-01
_C2 = 8.33331039e-03
_C3 = -1.98401553e-04
_C4 = 2.75294535e-06
_C5 = -2.46769610e-08
_C6 = 1.34514372e-10


def _sin(x):
    y = x * _INV2PI
    n = (y + _SMAGIC) - _SMAGIC
    r = x - n * _TPI_HI
    r = r - n * _TPI_LO
    r2 = r * r
    p = _C6
    p = p * r2 + _C5
    p = p * r2 + _C4
    p = p * r2 + _C3
    p = p * r2 + _C2
    p = p * r2 + _C1
    p = p * r2 + _C0
    return r * p


_MESH = plsc.VectorSubcoreMesh(core_axis_name="c", subcore_axis_name="s")


@functools.partial(
    pl.kernel,
    mesh=_MESH,
    compiler_params=pltpu.CompilerParams(needs_layout_passes=False),
    out_type=jax.ShapeDtypeStruct((2, NW, 16), jnp.float32),
    scratch_types=[
        pltpu.VMEM((CHUNK * 3,), jnp.float32),
        pltpu.VMEM((16,), jnp.float32),
        pltpu.VMEM((16,), jnp.float32),
    ],
)
def _reduce_kernel(attrs_hbm, part_hbm, buf, mscr, sscr):
    wid = lax.axis_index("s") * 2 + lax.axis_index("c")
    base = wid * EPW
    iota = lax.iota(jnp.int32, 16)
    idxf = iota * 3 + 1

    def chunk_body(c, carry):
        m, s = carry
        pltpu.sync_copy(attrs_hbm.at[pl.ds((base + c * CHUNK) * 3, CHUNK * 3)], buf)

        def gmax(g, cm):
            v = plsc.load_gather(buf, [idxf + g * 48])
            return jnp.maximum(cm, v)

        cm = lax.fori_loop(0, GROUPS, gmax, jnp.full((16,), -1e30, jnp.float32))
        mnew = jnp.maximum(m, cm)
        s = s * jnp.exp(m - mnew)

        def gsum(g, acc):
            v = plsc.load_gather(buf, [idxf + g * 48])
            return acc + jnp.exp(v - mnew)

        s = lax.fori_loop(0, GROUPS, gsum, s)
        return mnew, s

    m0 = jnp.full((16,), -1e30, jnp.float32)
    s0 = jnp.zeros((16,), jnp.float32)
    m, s = lax.fori_loop(0, NCHUNK, chunk_body, (m0, s0))
    mscr[...] = m
    sscr[...] = s
    pltpu.sync_copy(mscr, part_hbm.at[0, wid])
    pltpu.sync_copy(sscr, part_hbm.at[1, wid])


@functools.partial(
    pl.kernel,
    mesh=_MESH,
    compiler_params=pltpu.CompilerParams(needs_layout_passes=False),
    out_type=jax.ShapeDtypeStruct((E * OUT_D,), jnp.float32),
    scratch_types=[
        pltpu.VMEM((CHUNK * 3,), jnp.float32),
        pltpu.VMEM((CHUNK * OUT_D,), jnp.float32),
        pltpu.VMEM((2, NW, 16), jnp.float32),
        pltpu.VMEM((96, 16), jnp.float32),
    ],
)
def _main_kernel(attrs_hbm, consts_hbm, part_hbm, out_hbm, inb, outb, pv, cv):
    wid = lax.axis_index("s") * 2 + lax.axis_index("c")
    base = wid * EPW
    pltpu.sync_copy(part_hbm, pv)
    pltpu.sync_copy(consts_hbm, cv)

    m = pv[0, 0]
    s = pv[1, 0]
    for i in range(1, NW):
        mi = pv[0, i]
        si = pv[1, i]
        mn = jnp.maximum(m, mi)
        s = s * jnp.exp(m - mn) + si * jnp.exp(mi - mn)
        m = mn
    m_g = jnp.broadcast_to(jnp.max(m), (16,))
    s_g = jnp.broadcast_to(jnp.sum(s * jnp.exp(m - m_g)), (16,))
    inv_s = 1.0 / s_g

    iota = lax.iota(jnp.int32, 16)
    i3 = iota * 3
    i42 = iota * OUT_D
    w0 = cv[0]
    b0 = cv[1]
    wv = [cv[2 + k] for k in range(K)]
    bv = [cv[22 + k] for k in range(K)]
    e0v = [cv[42 + j] for j in range(K)]
    dev = [cv[62 + j] for j in range(K)]

    def chunk_body(c, carry):
        cbase = base + c * CHUNK
        pltpu.sync_copy(attrs_hbm.at[pl.ds(cbase * 3, CHUNK * 3)], inb)

        def group(g, carry2):
            gb = g * 48
            ty = plsc.load_gather(inb, [i3 + gb])
            fr = plsc.load_gather(inb, [i3 + (gb + 1)])
            t = plsc.load_gather(inb, [i3 + (gb + 2)])
            ob = g * (16 * OUT_D)
            plsc.store_scatter(outb, [i42 + ob], t * w0 + b0)
            for k in range(K):
                sv = _sin(t * wv[k] + bv[k])
                plsc.store_scatter(outb, [i42 + (ob + 1 + k)], sv)
            e = jnp.exp(fr - m_g) * inv_s
            wcol = jnp.where(ty == 1.0, 1.0, e)
            plsc.store_scatter(outb, [i42 + (ob + 21)], wcol)
            for j in range(K):
                col = ty * dev[j] + e0v[j]
                plsc.store_scatter(outb, [i42 + (ob + 22 + j)], col)
            return carry2

        lax.fori_loop(0, GROUPS, group, 0)
        pltpu.sync_copy(outb, out_hbm.at[pl.ds(cbase * OUT_D, CHUNK * OUT_D)])
        return carry

    lax.fori_loop(0, NCHUNK, chunk_body, 0)


def kernel(edge_attrs, t2v_w0, t2v_b0, t2v_w, t2v_b, type_emb):
    attrs_flat = edge_attrs.reshape(-1)
    demb = type_emb[1] - type_emb[0]
    consts = jnp.concatenate(
        [
            t2v_w0[None],
            t2v_b0[None],
            t2v_w,
            t2v_b,
            type_emb[0],
            demb,
            jnp.zeros((14,), jnp.float32),
        ]
    )
    consts = jnp.tile(consts[:, None], (1, 16))
    part = _reduce_kernel(attrs_flat)
    out = _main_kernel(attrs_flat, consts, part)
    return out.reshape(E, OUT_D)


# speedup vs baseline: 2.2749x; 2.2749x over previous
import functools

import jax
import jax.numpy as jnp
from jax import lax
from jax.experimental import pallas as pl
from jax.experimental.pallas import tpu as pltpu
from jax.experimental.pallas import tpu_sc as plsc

E = 1_600_000
K = 20
OUT_D = 42
NW = 32
EPW = E // NW
CHUNK = 400
NCHUNK = EPW // CHUNK
GROUPS = CHUNK // 16

_SMAGIC = float(1.5 * 2**23)
_INV2PI = 0.15915494309189535
_TPI_HI = 6.28125
_TPI_LO = 0.0028353071795864769
_C0 = 9.99999995e-01
_C1 = -1.66666646e-01
_C2 = 8.33331039e-03
_C3 = -1.98401553e-04
_C4 = 2.75294535e-06
_C5 = -2.46769610e-08
_C6 = 1.34514372e-10


def _sin(x):
    y = x * _INV2PI
    n = (y + _SMAGIC) - _SMAGIC
    r = x - n * _TPI_HI
    r = r - n * _TPI_LO
    r2 = r * r
    p = _C6
    p = p * r2 + _C5
    p = p * r2 + _C4
    p = p * r2 + _C3
    p = p * r2 + _C2
    p = p * r2 + _C1
    p = p * r2 + _C0
    return r * p


_MESH = plsc.VectorSubcoreMesh(core_axis_name="c", subcore_axis_name="s")


@functools.partial(
    pl.kernel,
    mesh=_MESH,
    compiler_params=pltpu.CompilerParams(needs_layout_passes=False),
    out_type=jax.ShapeDtypeStruct((2, NW, 16), jnp.float32),
    scratch_types=[
        pltpu.VMEM((CHUNK, 3), jnp.float32),
        pltpu.VMEM((16,), jnp.float32),
        pltpu.VMEM((16,), jnp.float32),
    ],
)
def _reduce_kernel(attrs_hbm, part_hbm, buf, mscr, sscr):
    wid = lax.axis_index("s") * 2 + lax.axis_index("c")
    base = wid * EPW
    iota = lax.iota(jnp.int32, 16)
    col1 = iota * 0 + 1

    def chunk_body(c, carry):
        m, s = carry
        pltpu.sync_copy(attrs_hbm.at[pl.ds(base + c * CHUNK, CHUNK)], buf)

        def gmax(g, cm):
            v = plsc.load_gather(buf, [iota + g * 16, col1])
            return jnp.maximum(cm, v)

        cm = lax.fori_loop(0, GROUPS, gmax, jnp.full((16,), -1e30, jnp.float32))
        mnew = jnp.maximum(m, cm)
        s = s * jnp.exp(m - mnew)

        def gsum(g, acc):
            v = plsc.load_gather(buf, [iota + g * 16, col1])
            return acc + jnp.exp(v - mnew)

        s = lax.fori_loop(0, GROUPS, gsum, s)
        return mnew, s

    m0 = jnp.full((16,), -1e30, jnp.float32)
    s0 = jnp.zeros((16,), jnp.float32)
    m, s = lax.fori_loop(0, NCHUNK, chunk_body, (m0, s0))
    mscr[...] = m
    sscr[...] = s
    pltpu.sync_copy(mscr, part_hbm.at[0, wid])
    pltpu.sync_copy(sscr, part_hbm.at[1, wid])


@functools.partial(
    pl.kernel,
    mesh=_MESH,
    compiler_params=pltpu.CompilerParams(needs_layout_passes=False),
    out_type=jax.ShapeDtypeStruct((E, OUT_D), jnp.float32),
    scratch_types=[
        pltpu.VMEM((CHUNK, 3), jnp.float32),
        pltpu.VMEM((CHUNK, OUT_D), jnp.float32),
        pltpu.VMEM((2, NW, 16), jnp.float32),
        pltpu.VMEM((96, 16), jnp.float32),
    ],
)
def _main_kernel(attrs_hbm, consts_hbm, part_hbm, out_hbm, inb, outb, pv, cv):
    wid = lax.axis_index("s") * 2 + lax.axis_index("c")
    base = wid * EPW
    pltpu.sync_copy(part_hbm, pv)
    pltpu.sync_copy(consts_hbm, cv)

    m = pv[0, 0]
    s = pv[1, 0]
    for i in range(1, NW):
        mi = pv[0, i]
        si = pv[1, i]
        mn = jnp.maximum(m, mi)
        s = s * jnp.exp(m - mn) + si * jnp.exp(mi - mn)
        m = mn
    m_g = jnp.broadcast_to(jnp.max(m), (16,))
    s_g = jnp.broadcast_to(jnp.sum(s * jnp.exp(m - m_g)), (16,))
    inv_s = 1.0 / s_g

    iota = lax.iota(jnp.int32, 16)
    zero = iota * 0
    cols = [zero + j for j in range(OUT_D)]
    w0 = cv[0]
    b0 = cv[1]
    wv = [cv[2 + k] for k in range(K)]
    bv = [cv[22 + k] for k in range(K)]
    e0v = [cv[42 + j] for j in range(K)]
    dev = [cv[62 + j] for j in range(K)]

    def chunk_body(c, carry):
        cbase = base + c * CHUNK
        pltpu.sync_copy(attrs_hbm.at[pl.ds(cbase, CHUNK)], inb)

        def group(g, carry2):
            rows = iota + g * 16
            ty = plsc.load_gather(inb, [rows, cols[0]])
            fr = plsc.load_gather(inb, [rows, cols[1]])
            t = plsc.load_gather(inb, [rows, cols[2]])
            plsc.store_scatter(outb, [rows, cols[0]], t * w0 + b0)
            for k in range(K):
                sv = _sin(t * wv[k] + bv[k])
                plsc.store_scatter(outb, [rows, cols[1 + k]], sv)
            e = jnp.exp(fr - m_g) * inv_s
            wcol = jnp.where(ty == 1.0, 1.0, e)
            plsc.store_scatter(outb, [rows, cols[21]], wcol)
            for j in range(K):
                col = ty * dev[j] + e0v[j]
                plsc.store_scatter(outb, [rows, cols[22 + j]], col)
            return carry2

        lax.fori_loop(0, GROUPS, group, 0)
        pltpu.sync_copy(outb, out_hbm.at[pl.ds(cbase, CHUNK)])
        return carry

    lax.fori_loop(0, NCHUNK, chunk_body, 0)


def kernel(edge_attrs, t2v_w0, t2v_b0, t2v_w, t2v_b, type_emb):
    demb = type_emb[1] - type_emb[0]
    consts = jnp.concatenate(
        [
            t2v_w0[None],
            t2v_b0[None],
            t2v_w,
            t2v_b,
            type_emb[0],
            demb,
            jnp.zeros((14,), jnp.float32),
        ]
    )
    consts = jnp.tile(consts[:, None], (1, 16))
    part = _reduce_kernel(edge_attrs)
    return _main_kernel(edge_attrs, consts, part)
